# baseline (speedup 1.0000x reference)
import jax
import jax.numpy as jnp
from jax import lax
from jax.experimental import pallas as pl
from jax.experimental.pallas import tpu as pltpu
from jax.experimental.pallas import tpu_sc as plsc

NC = 2
NS = 16
L = 16
H = 64
K = 400


def _mesh():
  return plsc.VectorSubcoreMesh(core_axis_name="c", subcore_axis_name="s")


def _params():
  return pltpu.CompilerParams(use_tc_tiling_on_sc=False,
                              needs_layout_passes=False)


def _fill(ref, value):
  v = jnp.full((L,), value, jnp.float32)
  if len(ref.shape) == 2:
    def body(r, _):
      for j in range(ref.shape[1] // L):
        ref[r, pl.ds(j * L, L)] = v
      return 0
    lax.fori_loop(0, ref.shape[0], body, 0)
  else:
    def body(r, _):
      ref[pl.ds(r * L, L)] = v
      return 0
    lax.fori_loop(0, ref.shape[0] // L, body, 0)


def _sqrt16(x):
  i = plsc.bitcast(x, jnp.int32)
  t = plsc.bitcast(jnp.int32(0x1FBD1DF5) + (i >> 1), jnp.float32)
  for _ in range(3):
    t = 0.5 * (t + x / t)
  return t


def _idx_transform(idx_ref, out_ref, c, n):
  def body(g, _):
    v = idx_ref[pl.ds(g * L, L)]
    out_ref[pl.ds(g * L, L)] = v + v + c
    return 0
  lax.fori_loop(0, n // L, body, 0)


def _halves_col(s0b, s1b, rid, j):
  if j < H:
    return plsc.load_gather(s0b, [rid, jnp.full((L,), j, jnp.int32)])
  return plsc.load_gather(s1b, [rid, jnp.full((L,), j - H, jnp.int32)])


def _make_scat(nseg, n_edges, mode, kblk):
  K = kblk
  nblocks = n_edges // K
  assert nblocks * K == n_edges and K % L == 0
  trips = -(-nblocks // NS)
  zrows = 20
  stripe_a = -(-(nseg // NS) // zrows) * zrows
  stripe_b = nseg - (NS - 1) * stripe_a
  assert stripe_b > 0 and stripe_b % zrows == 0

  outs = [jax.ShapeDtypeStruct((NC, nseg, H), jnp.float32)]
  if mode == 1:
    outs.append(jax.ShapeDtypeStruct((n_edges,), jnp.float32))

  scratch = [
      pltpu.VMEM((zrows, H), jnp.float32),
      pltpu.VMEM((K, H), jnp.float32),
      pltpu.VMEM((K,), jnp.int32),
      pltpu.VMEM((K,), jnp.int32),
      pltpu.VMEM((K,), jnp.int32),
  ]
  if mode != 0:
    scratch += [
        pltpu.VMEM((K,), jnp.float32),
        pltpu.VMEM((K,), jnp.float32),
        pltpu.VMEM((K,), jnp.float32),
    ]
  if mode == 2:
    scratch += [pltpu.VMEM((K,), jnp.float32)]
  scratch += [pltpu.VMEM_SHARED((nseg, H), jnp.float32)]
  scratch += [pltpu.SemaphoreType.DMA]

  def body(*args):
    pos = 0
    table = args[pos]; pos += 1
    irow = args[pos]; pos += 1
    iseg = args[pos]; pos += 1
    h = p = p_out = None
    sbuf = hbuf0 = hbuf1 = pbuf = None
    if mode in (1, 2, 3):
      h = args[pos]; pos += 1
    if mode == 2:
      p = args[pos]; pos += 1
    sums = args[pos]; pos += 1
    if mode == 1:
      p_out = args[pos]; pos += 1
    zbuf = args[pos]; pos += 1
    rows_v = args[pos]; pos += 1
    irow_v = args[pos]; pos += 1
    irow2_v = args[pos]; pos += 1
    iseg_v = args[pos]; pos += 1
    if mode != 0:
      sbuf = args[pos]; pos += 1
      hbuf0 = args[pos]; pos += 1
      hbuf1 = args[pos]; pos += 1
    if mode == 2:
      pbuf = args[pos]; pos += 1
    acc_sp = args[pos]; pos += 1
    sem = args[pos]

    c = lax.axis_index("c")
    s = lax.axis_index("s")

    _fill(zbuf, 0.0)

    @pl.when(s < NS - 1)
    def _():
      def zloop(z, _):
        pltpu.sync_copy(zbuf, acc_sp.at[pl.ds(s * stripe_a + z * zrows,
                                              zrows)])
        return 0
      lax.fori_loop(0, stripe_a // zrows, zloop, 0)

    @pl.when(s == NS - 1)
    def _():
      def zloop(z, _):
        pltpu.sync_copy(zbuf, acc_sp.at[pl.ds(s * stripe_a + z * zrows,
                                              zrows)])
        return 0
      lax.fori_loop(0, stripe_b // zrows, zloop, 0)
    plsc.subcore_barrier()

    row_iota = lax.iota(jnp.int32, L)

    def block(i, _):
      b = i * NS + s

      @pl.when(b < nblocks)
      def _():
        off = b * K
        pltpu.sync_copy(irow.at[pl.ds(off, K)], irow_v)
        pltpu.sync_copy(iseg.at[pl.ds(off, K)], iseg_v)
        _idx_transform(irow_v, irow2_v, c, K)
        pltpu.async_copy(table.at[irow2_v], rows_v, sem).wait()

        if mode in (1, 2, 3):
          pltpu.sync_copy(h.at[pl.ds(off, K)], hbuf0)
          pltpu.sync_copy(h.at[pl.ds(n_edges + off, K)], hbuf1)
          if mode == 2:
            pltpu.sync_copy(p.at[pl.ds(off, K)], pbuf)

          def mkscale(g, _):
            r = hbuf0[pl.ds(g * L, L)] + hbuf1[pl.ds(g * L, L)]
            if mode == 2:
              pv = pbuf[pl.ds(g * L, L)]
              r = pv * pv * r
            sbuf[pl.ds(g * L, L)] = r
            return 0
          lax.fori_loop(0, K // L, mkscale, 0)

          if mode == 1:
            @pl.when(c == 0)
            def _():
              pltpu.sync_copy(sbuf, p_out.at[pl.ds(off, K)])

          def scale_rows(g, _):
            rid = row_iota + g * L
            pv = sbuf[pl.ds(g * L, L)]
            for j in range(H):
              cid = jnp.full((L,), j, jnp.int32)
              v = plsc.load_gather(rows_v, [rid, cid])
              plsc.store_scatter(rows_v, [rid, cid], v * pv)
            return 0
          lax.fori_loop(0, K // L, scale_rows, 0)

        pltpu.sync_copy(rows_v, acc_sp.at[iseg_v], add=True)
      return 0
    lax.fori_loop(0, trips, block, 0)

    plsc.subcore_barrier()

    @pl.when(s < NS - 1)
    def _():
      pltpu.sync_copy(acc_sp.at[pl.ds(s * stripe_a, stripe_a)],
                      sums.at[c].at[pl.ds(s * stripe_a, stripe_a)])

    @pl.when(s == NS - 1)
    def _():
      pltpu.sync_copy(acc_sp.at[pl.ds(s * stripe_a, stripe_b)],
                      sums.at[c].at[pl.ds(s * stripe_a, stripe_b)])

  return pl.kernel(body, out_type=tuple(outs), mesh=_mesh(),
                   scratch_types=scratch, compiler_params=_params())


def _make_cnt(nseg, n_edges):
  nblocks = n_edges // K
  trips = -(-nblocks // NS)
  stripe_a = -(-(nseg // NS) // L) * L
  stripe_b = nseg - (NS - 1) * stripe_a
  assert stripe_b > 0 and stripe_b % L == 0

  scratch = [
      pltpu.VMEM((L, L), jnp.float32),
      pltpu.VMEM((K, L), jnp.float32),
      pltpu.VMEM((K,), jnp.int32),
      pltpu.VMEM((L,), jnp.float32),
      pltpu.VMEM_SHARED((nseg, L), jnp.float32),
      pltpu.SemaphoreType.DMA,
  ]

  def body(iseg, cnt_out, chb, onesb, iseg_v, resb, cnt16_sp, sem):
    c = lax.axis_index("c")
    s = lax.axis_index("s")
    row_iota = lax.iota(jnp.int32, L)
    stripe_n = stripe_a // L
    stripe_n_last = stripe_b // L

    @pl.when(c == 0)
    def _():
      _fill(chb, 0.0)
      _fill(onesb, 1.0)

      def zloop(q, _):
        pltpu.sync_copy(chb, cnt16_sp.at[pl.ds(s * stripe_a + q * L, L)])
        return 0
      lax.fori_loop(0, jnp.where(s == NS - 1, stripe_n_last, stripe_n),
                    zloop, 0)
    plsc.subcore_barrier()

    @pl.when(c == 0)
    def _():
      def block(i, _):
        b = i * NS + s

        @pl.when(b < nblocks)
        def _():
          off = b * K
          pltpu.sync_copy(iseg.at[pl.ds(off, K)], iseg_v)
          pltpu.sync_copy(onesb, cnt16_sp.at[iseg_v], add=True)
        return 0
      lax.fori_loop(0, trips, block, 0)
    plsc.subcore_barrier()

    @pl.when(c == 0)
    def _():
      def red(q, _):
        base = s * stripe_a + q * L
        pltpu.sync_copy(cnt16_sp.at[pl.ds(base, L)], chb)
        resb[...] = plsc.load_gather(
            chb, [row_iota, jnp.full((L,), 0, jnp.int32)])
        pltpu.sync_copy(resb, cnt_out.at[pl.ds(base, L)])
        return 0
      lax.fori_loop(0, jnp.where(s == NS - 1, stripe_n_last, stripe_n),
                    red, 0)

  return pl.kernel(body, out_type=jax.ShapeDtypeStruct((nseg,), jnp.float32),
                   mesh=_mesh(), scratch_types=scratch,
                   compiler_params=_params())


def _make_sim(n_edges):
  nblocks = n_edges // K
  trips = -(-nblocks // NS)

  scratch = [
      pltpu.VMEM((K, H), jnp.float32),
      pltpu.VMEM((K, H), jnp.float32),
      pltpu.VMEM((K,), jnp.int32),
      pltpu.VMEM((K,), jnp.int32),
      pltpu.VMEM((K,), jnp.int32),
      pltpu.VMEM((K,), jnp.float32),
      pltpu.SemaphoreType.DMA,
  ]

  def body(u2, table, iseg, irow, h_out,
           urows_v, trows_v, irow_v, irow2_v, iseg_v, hbuf, sem):
    c = lax.axis_index("c")
    s = lax.axis_index("s")
    row_iota = lax.iota(jnp.int32, L)

    def block(i, _):
      b = i * NS + s

      @pl.when(b < nblocks)
      def _():
        off = b * K
        pltpu.sync_copy(irow.at[pl.ds(off, K)], irow_v)
        pltpu.sync_copy(iseg.at[pl.ds(off, K)], iseg_v)
        _idx_transform(irow_v, irow2_v, c, K)
        _idx_transform(iseg_v, irow_v, c, K)
        cp1 = pltpu.async_copy(table.at[irow2_v], trows_v, sem)
        cp2 = pltpu.async_copy(u2.at[irow_v], urows_v, sem)
        cp1.wait()
        cp2.wait()

        def dots(g, _):
          rid = row_iota + g * L
          acc = jnp.zeros((L,), jnp.float32)
          for j in range(H):
            cid = jnp.full((L,), j, jnp.int32)
            a = plsc.load_gather(urows_v, [rid, cid])
            t = plsc.load_gather(trows_v, [rid, cid])
            acc = acc + a * t
          hbuf[pl.ds(g * L, L)] = acc
          return 0
        lax.fori_loop(0, K // L, dots, 0)

        pltpu.sync_copy(hbuf, h_out.at[pl.ds(c * n_edges + off, K)])
      return 0
    lax.fori_loop(0, trips, block, 0)

  return pl.kernel(
      body, out_type=jax.ShapeDtypeStruct((NC * n_edges,), jnp.float32),
      mesh=_mesh(), scratch_types=scratch, compiler_params=_params())


def _make_norm(nseg, nbase, squash):
  nblk = nseg // L
  trips = -(-nblk // (NC * NS))

  scratch = [
      pltpu.VMEM((L, H), jnp.float32),
      pltpu.VMEM((L, H), jnp.float32),
      pltpu.VMEM((L,), jnp.float32),
      pltpu.VMEM((L, 2 * H), jnp.float32),
      pltpu.VMEM((L, 2 * H), jnp.float32),
      pltpu.SemaphoreType.DMA,
  ]

  def body(sums, cnt, base, u_out, s0b, s1b, cntb, baseb, outb, sem):
    c = lax.axis_index("c")
    s = lax.axis_index("s")
    wid = s * NC + c
    rid = lax.iota(jnp.int32, L)

    def block(i, _):
      blk = i * (NC * NS) + wid

      @pl.when(blk < nblk)
      def _():
        s0 = blk * L
        base_off = lax.rem(s0, nbase)
        pltpu.sync_copy(sums.at[0].at[pl.ds(s0, L)], s0b)
        pltpu.sync_copy(sums.at[1].at[pl.ds(s0, L)], s1b)
        pltpu.sync_copy(cnt.at[pl.ds(s0, L)], cntb)
        pltpu.sync_copy(base.at[pl.ds(base_off, L)], baseb)
        inv = 1.0 / jnp.maximum(cntb[...], 1.0)
        if squash:
          nrm2 = jnp.zeros((L,), jnp.float32)
          for j in range(2 * H):
            mc = _halves_col(s0b, s1b, rid, j) * inv
            nrm2 = nrm2 + mc * mc
          fac = inv * (nrm2 / ((nrm2 + 1.0) *
                               jnp.maximum(_sqrt16(nrm2), 1e-12)))
        else:
          fac = inv
        for j in range(2 * H):
          cid = jnp.full((L,), j, jnp.int32)
          oc = (_halves_col(s0b, s1b, rid, j) * fac
                + plsc.load_gather(baseb, [rid, cid]))
          plsc.store_scatter(outb, [rid, cid], oc)
        pltpu.sync_copy(outb, u_out.at[pl.ds(s0, L)])
      return 0
    lax.fori_loop(0, trips, block, 0)

  return pl.kernel(
      body, out_type=jax.ShapeDtypeStruct((nseg, 2 * H), jnp.float32),
      mesh=_mesh(), scratch_types=scratch, compiler_params=_params())


def _make_mix(ne):
  nblk = ne // L
  trips = -(-nblk // (NC * NS))

  scratch = [
      pltpu.VMEM((3 * L, H), jnp.float32),
      pltpu.VMEM((3 * L, H), jnp.float32),
      pltpu.VMEM((3 * L,), jnp.float32),
      pltpu.VMEM((L, 2 * H), jnp.float32),
      pltpu.VMEM((L, 2 * H), jnp.float32),
      pltpu.VMEM((3 * L,), jnp.float32),
      pltpu.SemaphoreType.DMA,
  ]

  def body(sums, cnt, base, w48, agg_out, s0b, s1b, cntb, baseb, outb, wb,
           sem):
    c = lax.axis_index("c")
    s = lax.axis_index("s")
    wid = s * NC + c
    rid = lax.iota(jnp.int32, L)

    pltpu.sync_copy(w48, wb)
    esp = [jnp.exp(wb[pl.ds(r * L, L)]) for r in range(3)]
    denom = esp[0] + esp[1] + esp[2]
    sw = [e / denom for e in esp]

    def block(i, _):
      blk = i * (NC * NS) + wid

      @pl.when(blk < nblk)
      def _():
        i0 = blk * L
        for r in range(3):
          pltpu.sync_copy(sums.at[0].at[pl.ds(r * ne + i0, L)],
                          s0b.at[pl.ds(r * L, L)])
          pltpu.sync_copy(sums.at[1].at[pl.ds(r * ne + i0, L)],
                          s1b.at[pl.ds(r * L, L)])
          pltpu.sync_copy(cnt.at[pl.ds(r * ne + i0, L)],
                          cntb.at[pl.ds(r * L, L)])
        pltpu.sync_copy(base.at[pl.ds(i0, L)], baseb)
        fac = [sw[r] / jnp.maximum(cntb[pl.ds(r * L, L)], 1.0)
               for r in range(3)]
        for j in range(2 * H):
          cid = jnp.full((L,), j, jnp.int32)
          oc = plsc.load_gather(baseb, [rid, cid])
          for r in range(3):
            if j < H:
              v = plsc.load_gather(s0b, [rid + r * L,
                                         jnp.full((L,), j, jnp.int32)])
            else:
              v = plsc.load_gather(s1b, [rid + r * L,
                                         jnp.full((L,), j - H, jnp.int32)])
            oc = oc + v * fac[r]
          plsc.store_scatter(outb, [rid, cid], oc)
        pltpu.sync_copy(outb, agg_out.at[pl.ds(i0, L)])
      return 0
    lax.fori_loop(0, trips, block, 0)

  return pl.kernel(
      body, out_type=jax.ShapeDtypeStruct((ne, 2 * H), jnp.float32),
      mesh=_mesh(), scratch_types=scratch, compiler_params=_params())


def kernel(entity_emb, user_emb, edge_index, edge_type, user_index,
           item_index, w):
  ne = entity_emb.shape[0]
  nu = user_emb.shape[0]
  n_edges = edge_index.shape[1]
  nnz = user_index.shape[0]
  nseg = 3 * ne

  ent2 = entity_emb.reshape(2 * ne, H)
  head = edge_index[0]
  tail = edge_index[1]
  ch = edge_type * ne + head
  w48 = jnp.repeat(w, L)

  scat0 = _make_scat(nseg, n_edges, mode=0, kblk=80)
  scat1 = _make_scat(nseg, n_edges, mode=1, kblk=80)
  scat2 = _make_scat(nseg, n_edges, mode=2, kblk=80)
  sim_i = _make_sim(n_edges)
  norm_i = _make_norm(nseg, ne, squash=True)

  cnt = _make_cnt(nseg, n_edges)(ch)
  sums = scat0(ent2, tail, ch)[0]
  u = norm_i(sums, cnt, entity_emb)
  h = sim_i(u.reshape(2 * nseg, H), ent2, ch, tail)
  sums, p = scat1(ent2, tail, ch, h)
  u = norm_i(sums, cnt, entity_emb)
  h = sim_i(u.reshape(2 * nseg, H), ent2, ch, tail)
  sums = scat2(ent2, tail, ch, h, p)[0]
  entity_agg = _make_mix(ne)(sums, cnt, entity_emb, w48)

  scat0u = _make_scat(nu, nnz, mode=0, kblk=400)
  scat3u = _make_scat(nu, nnz, mode=3, kblk=400)
  sim_u = _make_sim(nnz)
  norm_u = _make_norm(nu, nu, squash=True)
  norm_uf = _make_norm(nu, nu, squash=False)

  cnt_u = _make_cnt(nu, nnz)(user_index)
  sums_u = scat0u(ent2, item_index, user_index)[0]
  uu = norm_u(sums_u, cnt_u, user_emb)
  h = sim_u(uu.reshape(2 * nu, H), ent2, user_index, item_index)
  sums_u = scat3u(ent2, item_index, user_index, h)[0]
  uu = norm_u(sums_u, cnt_u, user_emb)
  h = sim_u(uu.reshape(2 * nu, H), ent2, user_index, item_index)
  sums_u = scat3u(ent2, item_index, user_index, h)[0]
  user_agg = norm_uf(sums_u, cnt_u, user_emb)

  return (entity_agg, user_agg)


# speedup vs baseline: 2.5433x; 2.5433x over previous
import jax
import jax.numpy as jnp
from jax import lax
from jax.experimental import pallas as pl
from jax.experimental.pallas import tpu as pltpu
from jax.experimental.pallas import tpu_sc as plsc

NC = 2
NS = 16
L = 16
H = 64
K = 400


def _mesh():
  return plsc.VectorSubcoreMesh(core_axis_name="c", subcore_axis_name="s")


def _params():
  return pltpu.CompilerParams(use_tc_tiling_on_sc=False,
                              needs_layout_passes=False)


def _fill(ref, value):
  v = jnp.full((L,), value, jnp.float32)
  if len(ref.shape) == 2:
    def body(r, _):
      for j in range(ref.shape[1] // L):
        ref[r, pl.ds(j * L, L)] = v
      return 0
    lax.fori_loop(0, ref.shape[0], body, 0)
  else:
    def body(r, _):
      ref[pl.ds(r * L, L)] = v
      return 0
    lax.fori_loop(0, ref.shape[0] // L, body, 0)


def _sqrt16(x):
  i = plsc.bitcast(x, jnp.int32)
  t = plsc.bitcast(jnp.int32(0x1FBD1DF5) + (i >> 1), jnp.float32)
  for _ in range(3):
    t = 0.5 * (t + x / t)
  return t


def _idx_transform(idx_ref, out_ref, c, n):
  def body(g, _):
    v = idx_ref[pl.ds(g * L, L)]
    out_ref[pl.ds(g * L, L)] = v + v + c
    return 0
  lax.fori_loop(0, n // L, body, 0)


def _make_scat(nseg, n_edges, mode, kblk):
  K = kblk
  nblocks = n_edges // K
  assert nblocks * K == n_edges and K % L == 0
  trips = -(-nblocks // NS)
  zrows = 20
  stripe_a = -(-(nseg // NS) // zrows) * zrows
  stripe_b = nseg - (NS - 1) * stripe_a
  assert stripe_b > 0 and stripe_b % zrows == 0

  outs = [jax.ShapeDtypeStruct((NC, nseg, H), jnp.float32)]
  if mode == 1:
    outs.append(jax.ShapeDtypeStruct((n_edges,), jnp.float32))

  scratch = [
      pltpu.VMEM((zrows, H), jnp.float32),
      pltpu.VMEM((K, H), jnp.float32),
      pltpu.VMEM((K,), jnp.int32),
      pltpu.VMEM((K,), jnp.int32),
      pltpu.VMEM((K,), jnp.int32),
  ]
  if mode != 0:
    scratch += [
        pltpu.VMEM((K,), jnp.float32),
        pltpu.VMEM((K,), jnp.float32),
        pltpu.VMEM((K,), jnp.float32),
    ]
  if mode == 2:
    scratch += [pltpu.VMEM((K,), jnp.float32)]
  scratch += [pltpu.VMEM_SHARED((nseg, H), jnp.float32)]
  scratch += [pltpu.SemaphoreType.DMA]

  def body(*args):
    pos = 0
    table = args[pos]; pos += 1
    irow = args[pos]; pos += 1
    iseg = args[pos]; pos += 1
    h = p = p_out = None
    sbuf = hbuf0 = hbuf1 = pbuf = None
    if mode in (1, 2, 3):
      h = args[pos]; pos += 1
    if mode == 2:
      p = args[pos]; pos += 1
    sums = args[pos]; pos += 1
    if mode == 1:
      p_out = args[pos]; pos += 1
    zbuf = args[pos]; pos += 1
    rows_v = args[pos]; pos += 1
    irow_v = args[pos]; pos += 1
    irow2_v = args[pos]; pos += 1
    iseg_v = args[pos]; pos += 1
    if mode != 0:
      sbuf = args[pos]; pos += 1
      hbuf0 = args[pos]; pos += 1
      hbuf1 = args[pos]; pos += 1
    if mode == 2:
      pbuf = args[pos]; pos += 1
    acc_sp = args[pos]; pos += 1
    sem = args[pos]

    c = lax.axis_index("c")
    s = lax.axis_index("s")

    _fill(zbuf, 0.0)

    @pl.when(s < NS - 1)
    def _():
      def zloop(z, _):
        pltpu.sync_copy(zbuf, acc_sp.at[pl.ds(s * stripe_a + z * zrows,
                                              zrows)])
        return 0
      lax.fori_loop(0, stripe_a // zrows, zloop, 0)

    @pl.when(s == NS - 1)
    def _():
      def zloop(z, _):
        pltpu.sync_copy(zbuf, acc_sp.at[pl.ds(s * stripe_a + z * zrows,
                                              zrows)])
        return 0
      lax.fori_loop(0, stripe_b // zrows, zloop, 0)
    plsc.subcore_barrier()

    row_iota = lax.iota(jnp.int32, L)

    def block(i, _):
      b = i * NS + s

      @pl.when(b < nblocks)
      def _():
        off = b * K
        pltpu.sync_copy(irow.at[pl.ds(off, K)], irow_v)
        pltpu.sync_copy(iseg.at[pl.ds(off, K)], iseg_v)
        _idx_transform(irow_v, irow2_v, c, K)
        pltpu.async_copy(table.at[irow2_v], rows_v, sem).wait()

        if mode in (1, 2, 3):
          pltpu.sync_copy(h.at[pl.ds(off, K)], hbuf0)
          pltpu.sync_copy(h.at[pl.ds(n_edges + off, K)], hbuf1)
          if mode == 2:
            pltpu.sync_copy(p.at[pl.ds(off, K)], pbuf)

          def mkscale(g, _):
            r = hbuf0[pl.ds(g * L, L)] + hbuf1[pl.ds(g * L, L)]
            if mode == 2:
              pv = pbuf[pl.ds(g * L, L)]
              r = pv * pv * r
            sbuf[pl.ds(g * L, L)] = r
            return 0
          lax.fori_loop(0, K // L, mkscale, 0)

          if mode == 1:
            @pl.when(c == 0)
            def _():
              pltpu.sync_copy(sbuf, p_out.at[pl.ds(off, K)])

          def scale_rows(g, _):
            rid = row_iota + g * L
            pv = sbuf[pl.ds(g * L, L)]
            for j in range(H):
              cid = (rid + j) & (H - 1)
              v = plsc.load_gather(rows_v, [rid, cid])
              plsc.store_scatter(rows_v, [rid, cid], v * pv)
            return 0
          lax.fori_loop(0, K // L, scale_rows, 0)

        pltpu.sync_copy(rows_v, acc_sp.at[iseg_v], add=True)
      return 0
    lax.fori_loop(0, trips, block, 0)

    plsc.subcore_barrier()

    @pl.when(s < NS - 1)
    def _():
      pltpu.sync_copy(acc_sp.at[pl.ds(s * stripe_a, stripe_a)],
                      sums.at[c].at[pl.ds(s * stripe_a, stripe_a)])

    @pl.when(s == NS - 1)
    def _():
      pltpu.sync_copy(acc_sp.at[pl.ds(s * stripe_a, stripe_b)],
                      sums.at[c].at[pl.ds(s * stripe_a, stripe_b)])

  return pl.kernel(body, out_type=tuple(outs), mesh=_mesh(),
                   scratch_types=scratch, compiler_params=_params())


def _make_cnt(nseg, n_edges, kblk=800):
  K = kblk
  nblocks = n_edges // K
  trips = -(-nblocks // NS)
  stripe_a = -(-(nseg // NS) // L) * L
  stripe_b = nseg - (NS - 1) * stripe_a
  assert stripe_b > 0 and stripe_b % L == 0

  scratch = [
      pltpu.VMEM((L, L), jnp.float32),
      pltpu.VMEM((K, L), jnp.float32),
      pltpu.VMEM((K,), jnp.int32),
      pltpu.VMEM((L,), jnp.float32),
      pltpu.VMEM_SHARED((nseg, L), jnp.float32),
      pltpu.SemaphoreType.DMA,
  ]

  def body(iseg, cnt_out, chb, onesb, iseg_v, resb, cnt16_sp, sem):
    c = lax.axis_index("c")
    s = lax.axis_index("s")
    row_iota = lax.iota(jnp.int32, L)
    stripe_n = stripe_a // L
    stripe_n_last = stripe_b // L

    @pl.when(c == 0)
    def _():
      _fill(chb, 0.0)
      _fill(onesb, 1.0)

      def zloop(q, _):
        pltpu.sync_copy(chb, cnt16_sp.at[pl.ds(s * stripe_a + q * L, L)])
        return 0
      lax.fori_loop(0, jnp.where(s == NS - 1, stripe_n_last, stripe_n),
                    zloop, 0)
    plsc.subcore_barrier()

    @pl.when(c == 0)
    def _():
      def block(i, _):
        b = i * NS + s

        @pl.when(b < nblocks)
        def _():
          off = b * K
          pltpu.sync_copy(iseg.at[pl.ds(off, K)], iseg_v)
          pltpu.sync_copy(onesb, cnt16_sp.at[iseg_v], add=True)
        return 0
      lax.fori_loop(0, trips, block, 0)
    plsc.subcore_barrier()

    @pl.when(c == 0)
    def _():
      def red(q, _):
        base = s * stripe_a + q * L
        pltpu.sync_copy(cnt16_sp.at[pl.ds(base, L)], chb)
        resb[...] = plsc.load_gather(chb, [row_iota, row_iota])
        pltpu.sync_copy(resb, cnt_out.at[pl.ds(base, L)])
        return 0
      lax.fori_loop(0, jnp.where(s == NS - 1, stripe_n_last, stripe_n),
                    red, 0)

  return pl.kernel(body, out_type=jax.ShapeDtypeStruct((nseg,), jnp.float32),
                   mesh=_mesh(), scratch_types=scratch,
                   compiler_params=_params())


def _make_sim(n_edges, kblk=800):
  K = kblk
  nblocks = n_edges // K
  trips = -(-nblocks // NS)

  scratch = [
      pltpu.VMEM((K, H), jnp.float32),
      pltpu.VMEM((K, H), jnp.float32),
      pltpu.VMEM((K,), jnp.int32),
      pltpu.VMEM((K,), jnp.int32),
      pltpu.VMEM((K,), jnp.int32),
      pltpu.VMEM((K,), jnp.float32),
      pltpu.SemaphoreType.DMA,
  ]

  def body(u2, table, iseg, irow, h_out,
           urows_v, trows_v, irow_v, irow2_v, iseg_v, hbuf, sem):
    c = lax.axis_index("c")
    s = lax.axis_index("s")
    row_iota = lax.iota(jnp.int32, L)

    def block(i, _):
      b = i * NS + s

      @pl.when(b < nblocks)
      def _():
        off = b * K
        pltpu.sync_copy(irow.at[pl.ds(off, K)], irow_v)
        pltpu.sync_copy(iseg.at[pl.ds(off, K)], iseg_v)
        _idx_transform(irow_v, irow2_v, c, K)
        _idx_transform(iseg_v, irow_v, c, K)
        cp1 = pltpu.async_copy(table.at[irow2_v], trows_v, sem)
        cp2 = pltpu.async_copy(u2.at[irow_v], urows_v, sem)
        cp1.wait()
        cp2.wait()

        def dots(g, _):
          rid = row_iota + g * L
          acc = jnp.zeros((L,), jnp.float32)
          for j in range(H):
            cid = (rid + j) & (H - 1)
            a = plsc.load_gather(urows_v, [rid, cid])
            t = plsc.load_gather(trows_v, [rid, cid])
            acc = acc + a * t
          hbuf[pl.ds(g * L, L)] = acc
          return 0
        lax.fori_loop(0, K // L, dots, 0)

        pltpu.sync_copy(hbuf, h_out.at[pl.ds(c * n_edges + off, K)])
      return 0
    lax.fori_loop(0, trips, block, 0)

  return pl.kernel(
      body, out_type=jax.ShapeDtypeStruct((NC * n_edges,), jnp.float32),
      mesh=_mesh(), scratch_types=scratch, compiler_params=_params())


def _make_norm(nseg, nbase, squash):
  nblk = nseg // L
  trips = -(-nblk // (NC * NS))

  scratch = [
      pltpu.VMEM((L, H), jnp.float32),
      pltpu.VMEM((L, H), jnp.float32),
      pltpu.VMEM((L,), jnp.float32),
      pltpu.VMEM((L, 2 * H), jnp.float32),
      pltpu.VMEM((L, 2 * H), jnp.float32),
      pltpu.SemaphoreType.DMA,
  ]

  def body(sums, cnt, base, u_out, s0b, s1b, cntb, baseb, outb, sem):
    c = lax.axis_index("c")
    s = lax.axis_index("s")
    wid = s * NC + c
    rid = lax.iota(jnp.int32, L)

    def block(i, _):
      blk = i * (NC * NS) + wid

      @pl.when(blk < nblk)
      def _():
        s0 = blk * L
        base_off = lax.rem(s0, nbase)
        pltpu.sync_copy(sums.at[0].at[pl.ds(s0, L)], s0b)
        pltpu.sync_copy(sums.at[1].at[pl.ds(s0, L)], s1b)
        pltpu.sync_copy(cnt.at[pl.ds(s0, L)], cntb)
        pltpu.sync_copy(base.at[pl.ds(base_off, L)], baseb)
        inv = 1.0 / jnp.maximum(cntb[...], 1.0)
        if squash:
          nrm2 = jnp.zeros((L,), jnp.float32)
          for j in range(H):
            ca = (rid + j) & (H - 1)
            m0 = plsc.load_gather(s0b, [rid, ca]) * inv
            m1 = plsc.load_gather(s1b, [rid, ca]) * inv
            nrm2 = nrm2 + m0 * m0 + m1 * m1
          fac = inv * (nrm2 / ((nrm2 + 1.0) *
                               jnp.maximum(_sqrt16(nrm2), 1e-12)))
        else:
          fac = inv
        for j in range(H):
          ca = (rid + j) & (H - 1)
          cb = ca + H
          v0 = (plsc.load_gather(s0b, [rid, ca]) * fac
                + plsc.load_gather(baseb, [rid, ca]))
          plsc.store_scatter(outb, [rid, ca], v0)
          v1 = (plsc.load_gather(s1b, [rid, ca]) * fac
                + plsc.load_gather(baseb, [rid, cb]))
          plsc.store_scatter(outb, [rid, cb], v1)
        pltpu.sync_copy(outb, u_out.at[pl.ds(s0, L)])
      return 0
    lax.fori_loop(0, trips, block, 0)

  return pl.kernel(
      body, out_type=jax.ShapeDtypeStruct((nseg, 2 * H), jnp.float32),
      mesh=_mesh(), scratch_types=scratch, compiler_params=_params())


def _make_mix(ne):
  nblk = ne // L
  trips = -(-nblk // (NC * NS))

  scratch = [
      pltpu.VMEM((3 * L, H), jnp.float32),
      pltpu.VMEM((3 * L, H), jnp.float32),
      pltpu.VMEM((3 * L,), jnp.float32),
      pltpu.VMEM((L, 2 * H), jnp.float32),
      pltpu.VMEM((L, 2 * H), jnp.float32),
      pltpu.VMEM((3 * L,), jnp.float32),
      pltpu.SemaphoreType.DMA,
  ]

  def body(sums, cnt, base, w48, agg_out, s0b, s1b, cntb, baseb, outb, wb,
           sem):
    c = lax.axis_index("c")
    s = lax.axis_index("s")
    wid = s * NC + c
    rid = lax.iota(jnp.int32, L)

    pltpu.sync_copy(w48, wb)
    esp = [jnp.exp(wb[pl.ds(r * L, L)]) for r in range(3)]
    denom = esp[0] + esp[1] + esp[2]
    sw = [e / denom for e in esp]

    def block(i, _):
      blk = i * (NC * NS) + wid

      @pl.when(blk < nblk)
      def _():
        i0 = blk * L
        for r in range(3):
          pltpu.sync_copy(sums.at[0].at[pl.ds(r * ne + i0, L)],
                          s0b.at[pl.ds(r * L, L)])
          pltpu.sync_copy(sums.at[1].at[pl.ds(r * ne + i0, L)],
                          s1b.at[pl.ds(r * L, L)])
          pltpu.sync_copy(cnt.at[pl.ds(r * ne + i0, L)],
                          cntb.at[pl.ds(r * L, L)])
        pltpu.sync_copy(base.at[pl.ds(i0, L)], baseb)
        fac = [sw[r] / jnp.maximum(cntb[pl.ds(r * L, L)], 1.0)
               for r in range(3)]
        for j in range(H):
          ca = (rid + j) & (H - 1)
          cb = ca + H
          oc0 = plsc.load_gather(baseb, [rid, ca])
          oc1 = plsc.load_gather(baseb, [rid, cb])
          for r in range(3):
            oc0 = oc0 + plsc.load_gather(s0b, [rid + r * L, ca]) * fac[r]
            oc1 = oc1 + plsc.load_gather(s1b, [rid + r * L, ca]) * fac[r]
          plsc.store_scatter(outb, [rid, ca], oc0)
          plsc.store_scatter(outb, [rid, cb], oc1)
        pltpu.sync_copy(outb, agg_out.at[pl.ds(i0, L)])
      return 0
    lax.fori_loop(0, trips, block, 0)

  return pl.kernel(
      body, out_type=jax.ShapeDtypeStruct((ne, 2 * H), jnp.float32),
      mesh=_mesh(), scratch_types=scratch, compiler_params=_params())


def kernel(entity_emb, user_emb, edge_index, edge_type, user_index,
           item_index, w):
  ne = entity_emb.shape[0]
  nu = user_emb.shape[0]
  n_edges = edge_index.shape[1]
  nnz = user_index.shape[0]
  nseg = 3 * ne

  ent2 = entity_emb.reshape(2 * ne, H)
  head = edge_index[0]
  tail = edge_index[1]
  ch = edge_type * ne + head
  w48 = jnp.repeat(w, L)

  scat0 = _make_scat(nseg, n_edges, mode=0, kblk=80)
  scat1 = _make_scat(nseg, n_edges, mode=1, kblk=80)
  scat2 = _make_scat(nseg, n_edges, mode=2, kblk=80)
  sim_i = _make_sim(n_edges)
  norm_i = _make_norm(nseg, ne, squash=True)

  cnt = _make_cnt(nseg, n_edges)(ch)
  sums = scat0(ent2, tail, ch)[0]
  u = norm_i(sums, cnt, entity_emb)
  h = sim_i(u.reshape(2 * nseg, H), ent2, ch, tail)
  sums, p = scat1(ent2, tail, ch, h)
  u = norm_i(sums, cnt, entity_emb)
  h = sim_i(u.reshape(2 * nseg, H), ent2, ch, tail)
  sums = scat2(ent2, tail, ch, h, p)[0]
  entity_agg = _make_mix(ne)(sums, cnt, entity_emb, w48)

  scat0u = _make_scat(nu, nnz, mode=0, kblk=400)
  scat3u = _make_scat(nu, nnz, mode=3, kblk=400)
  sim_u = _make_sim(nnz)
  norm_u = _make_norm(nu, nu, squash=True)
  norm_uf = _make_norm(nu, nu, squash=False)

  cnt_u = _make_cnt(nu, nnz)(user_index)
  sums_u = scat0u(ent2, item_index, user_index)[0]
  uu = norm_u(sums_u, cnt_u, user_emb)
  h = sim_u(uu.reshape(2 * nu, H), ent2, user_index, item_index)
  sums_u = scat3u(ent2, item_index, user_index, h)[0]
  uu = norm_u(sums_u, cnt_u, user_emb)
  h = sim_u(uu.reshape(2 * nu, H), ent2, user_index, item_index)
  sums_u = scat3u(ent2, item_index, user_index, h)[0]
  user_agg = norm_uf(sums_u, cnt_u, user_emb)

  return (entity_agg, user_agg)
